# Initial kernel scaffold; baseline (speedup 1.0000x reference)
import jax
import jax.numpy as jnp
from jax.experimental import pallas as pl


def kernel(query_features, W_attn, b_attn, W_ffn, b_ffn, E_attn_vis, E_attn_text, E_attn_general, E_ffn_vis, E_ffn_text, E_ffn_general):
    raise NotImplementedError("write your pallas kernel here")


# speedup vs baseline: 3.1614x; 3.1614x over previous
import functools

import jax
import jax.numpy as jnp
from jax.experimental import pallas as pl
from jax.experimental.pallas import tpu as pltpu

B, QD, AD, FD = 4096, 4096, 4096, 16384
BLK = 256
NE = 8


def _top2_softmax(l):
    col = jax.lax.broadcasted_iota(jnp.int32, l.shape, 1)
    m1 = jnp.max(l, axis=-1, keepdims=True)
    i1 = jnp.min(jnp.where(l == m1, col, NE), axis=-1, keepdims=True)
    l2 = jnp.where(col == i1, -jnp.inf, l)
    m2 = jnp.max(l2, axis=-1, keepdims=True)
    i2 = jnp.min(jnp.where(l2 == m2, col, NE), axis=-1, keepdims=True)
    s = jnp.exp(m2 - m1)
    w1 = 1.0 / (1.0 + s)
    w2 = 1.0 - w1
    return jnp.where(col == i1, w1, 0.0) + jnp.where(col == i2, w2, 0.0)


def _body(x_ref, w_ref, b_ref, ea_ref, ef_ref,
          la_ref, lf_ref, va_ref, vf_ref):
    x = x_ref[...]
    logits = jax.lax.dot_general(
        x, w_ref[...], (((1,), (1,)), ((), ())),
        preferred_element_type=jnp.float32) + b_ref[...]
    la = logits[:, :NE]
    lf = logits[:, NE:]
    la_ref[...] = la
    lf_ref[...] = lf
    ones = jnp.ones((x.shape[0], 1), jnp.float32)
    wa = jnp.concatenate([_top2_softmax(la), ones], axis=1)
    wf = jnp.concatenate([_top2_softmax(lf), ones], axis=1)
    va_ref[...] = jax.lax.dot_general(
        wa, ea_ref[...], (((1,), (0,)), ((), ())),
        preferred_element_type=jnp.float32)
    vf_ref[...] = jax.lax.dot_general(
        wf, ef_ref[...], (((1,), (0,)), ((), ())),
        preferred_element_type=jnp.float32)


@jax.jit
def kernel(query_features, W_attn, b_attn, W_ffn, b_ffn,
           E_attn_vis, E_attn_text, E_attn_general,
           E_ffn_vis, E_ffn_text, E_ffn_general):
    w = jnp.concatenate([W_attn, W_ffn], axis=0)
    b = jnp.concatenate([b_attn, b_ffn])[None, :]
    ea = jnp.concatenate([E_attn_vis, E_attn_text, E_attn_general], axis=0)
    ef = jnp.concatenate([E_ffn_vis, E_ffn_text, E_ffn_general], axis=0)

    grid = (B // BLK,)
    la, lf, va, vf = pl.pallas_call(
        _body,
        grid=grid,
        in_specs=[
            pl.BlockSpec((BLK, QD), lambda i: (i, 0)),
            pl.BlockSpec((16, QD), lambda i: (0, 0)),
            pl.BlockSpec((1, 16), lambda i: (0, 0)),
            pl.BlockSpec((9, AD), lambda i: (0, 0)),
            pl.BlockSpec((9, FD), lambda i: (0, 0)),
        ],
        out_specs=[
            pl.BlockSpec((BLK, NE), lambda i: (i, 0)),
            pl.BlockSpec((BLK, NE), lambda i: (i, 0)),
            pl.BlockSpec((BLK, AD), lambda i: (i, 0)),
            pl.BlockSpec((BLK, FD), lambda i: (i, 0)),
        ],
        out_shape=[
            jax.ShapeDtypeStruct((B, NE), jnp.float32),
            jax.ShapeDtypeStruct((B, NE), jnp.float32),
            jax.ShapeDtypeStruct((B, AD), jnp.float32),
            jax.ShapeDtypeStruct((B, FD), jnp.float32),
        ],
        compiler_params=pltpu.CompilerParams(
            dimension_semantics=("arbitrary",),
        ),
    )(query_features, w, b, ea, ef)
    return (va, vf, la, lf)


# speedup vs baseline: 3.1732x; 1.0037x over previous
import functools

import jax
import jax.numpy as jnp
from jax.experimental import pallas as pl
from jax.experimental.pallas import tpu as pltpu

B, QD, AD, FD = 4096, 4096, 4096, 16384
BLK = 256
NE = 8


def _top2_softmax(l):
    col = jax.lax.broadcasted_iota(jnp.int32, l.shape, 1)
    m1 = jnp.max(l, axis=-1, keepdims=True)
    i1 = jnp.min(jnp.where(l == m1, col, NE), axis=-1, keepdims=True)
    l2 = jnp.where(col == i1, -jnp.inf, l)
    m2 = jnp.max(l2, axis=-1, keepdims=True)
    i2 = jnp.min(jnp.where(l2 == m2, col, NE), axis=-1, keepdims=True)
    s = jnp.exp(m2 - m1)
    w1 = 1.0 / (1.0 + s)
    w2 = 1.0 - w1
    return jnp.where(col == i1, w1, 0.0) + jnp.where(col == i2, w2, 0.0)


def _body(x_ref, w_ref, b_ref, ea_ref, ef_ref,
          la_ref, lf_ref, va_ref, vf_ref):
    x = x_ref[...]
    logits = jax.lax.dot_general(
        x, w_ref[...], (((1,), (1,)), ((), ())),
        preferred_element_type=jnp.float32) + b_ref[...]
    la = logits[:, :NE]
    lf = logits[:, NE:]
    la_ref[...] = la
    lf_ref[...] = lf
    ones = jnp.ones((x.shape[0], 1), jnp.float32)
    wa = jnp.concatenate([_top2_softmax(la), ones], axis=1)
    wf = jnp.concatenate([_top2_softmax(lf), ones], axis=1)
    va_ref[...] = jax.lax.dot_general(
        wa, ea_ref[...], (((1,), (0,)), ((), ())),
        preferred_element_type=jnp.float32)
    vf_ref[...] = jax.lax.dot_general(
        wf, ef_ref[...], (((1,), (0,)), ((), ())),
        preferred_element_type=jnp.float32)


@jax.jit
def kernel(query_features, W_attn, b_attn, W_ffn, b_ffn,
           E_attn_vis, E_attn_text, E_attn_general,
           E_ffn_vis, E_ffn_text, E_ffn_general):
    w = jnp.concatenate([W_attn, W_ffn], axis=0)
    b = jnp.concatenate([b_attn, b_ffn])[None, :]
    ea = jnp.concatenate([E_attn_vis, E_attn_text, E_attn_general], axis=0)
    ef = jnp.concatenate([E_ffn_vis, E_ffn_text, E_ffn_general], axis=0)

    grid = (B // BLK,)
    la, lf, va, vf = pl.pallas_call(
        _body,
        grid=grid,
        in_specs=[
            pl.BlockSpec((BLK, QD), lambda i: (i, 0)),
            pl.BlockSpec((16, QD), lambda i: (0, 0)),
            pl.BlockSpec((1, 16), lambda i: (0, 0)),
            pl.BlockSpec((9, AD), lambda i: (0, 0)),
            pl.BlockSpec((9, FD), lambda i: (0, 0)),
        ],
        out_specs=[
            pl.BlockSpec((BLK, NE), lambda i: (i, 0)),
            pl.BlockSpec((BLK, NE), lambda i: (i, 0)),
            pl.BlockSpec((BLK, AD), lambda i: (i, 0)),
            pl.BlockSpec((BLK, FD), lambda i: (i, 0)),
        ],
        out_shape=[
            jax.ShapeDtypeStruct((B, NE), jnp.float32),
            jax.ShapeDtypeStruct((B, NE), jnp.float32),
            jax.ShapeDtypeStruct((B, AD), jnp.float32),
            jax.ShapeDtypeStruct((B, FD), jnp.float32),
        ],
        compiler_params=pltpu.CompilerParams(
            dimension_semantics=("parallel",),
        ),
    )(query_features, w, b, ea, ef)
    return (va, vf, la, lf)


# speedup vs baseline: 3.5857x; 1.1300x over previous
import functools

import jax
import jax.numpy as jnp
import numpy as np
from jax.experimental import pallas as pl
from jax.experimental.pallas import tpu as pltpu
from jax.sharding import Mesh, PartitionSpec as P

B, QD, AD, FD = 4096, 4096, 4096, 16384
BLK = 256
NE = 8


def _top2_softmax(l):
    col = jax.lax.broadcasted_iota(jnp.int32, l.shape, 1)
    m1 = jnp.max(l, axis=-1, keepdims=True)
    i1 = jnp.min(jnp.where(l == m1, col, NE), axis=-1, keepdims=True)
    l2 = jnp.where(col == i1, -jnp.inf, l)
    m2 = jnp.max(l2, axis=-1, keepdims=True)
    i2 = jnp.min(jnp.where(l2 == m2, col, NE), axis=-1, keepdims=True)
    s = jnp.exp(m2 - m1)
    w1 = 1.0 / (1.0 + s)
    w2 = 1.0 - w1
    return jnp.where(col == i1, w1, 0.0) + jnp.where(col == i2, w2, 0.0)


def _body(x_ref, w_ref, b_ref, ea_ref, ef_ref,
          la_ref, lf_ref, va_ref, vf_ref):
    x = x_ref[...]
    logits = jax.lax.dot_general(
        x, w_ref[...], (((1,), (1,)), ((), ())),
        preferred_element_type=jnp.float32) + b_ref[...]
    la = logits[:, :NE]
    lf = logits[:, NE:]
    la_ref[...] = la
    lf_ref[...] = lf
    ones = jnp.ones((x.shape[0], 1), jnp.float32)
    wa = jnp.concatenate([_top2_softmax(la), ones], axis=1)
    wf = jnp.concatenate([_top2_softmax(lf), ones], axis=1)
    va_ref[...] = jax.lax.dot_general(
        wa, ea_ref[...], (((1,), (0,)), ((), ())),
        preferred_element_type=jnp.float32)
    vf_ref[...] = jax.lax.dot_general(
        wf, ef_ref[...], (((1,), (0,)), ((), ())),
        preferred_element_type=jnp.float32)


def _run_shard(x, w, b, ea, ef):
    nb = x.shape[0]
    grid = (nb // BLK,)
    la, lf, va, vf = pl.pallas_call(
        _body,
        grid=grid,
        in_specs=[
            pl.BlockSpec((BLK, QD), lambda i: (i, 0)),
            pl.BlockSpec((16, QD), lambda i: (0, 0)),
            pl.BlockSpec((1, 16), lambda i: (0, 0)),
            pl.BlockSpec((9, AD), lambda i: (0, 0)),
            pl.BlockSpec((9, FD), lambda i: (0, 0)),
        ],
        out_specs=[
            pl.BlockSpec((BLK, NE), lambda i: (i, 0)),
            pl.BlockSpec((BLK, NE), lambda i: (i, 0)),
            pl.BlockSpec((BLK, AD), lambda i: (i, 0)),
            pl.BlockSpec((BLK, FD), lambda i: (i, 0)),
        ],
        out_shape=[
            jax.ShapeDtypeStruct((nb, NE), jnp.float32),
            jax.ShapeDtypeStruct((nb, NE), jnp.float32),
            jax.ShapeDtypeStruct((nb, AD), jnp.float32),
            jax.ShapeDtypeStruct((nb, FD), jnp.float32),
        ],
        compiler_params=pltpu.CompilerParams(
            dimension_semantics=("arbitrary",),
        ),
    )(x, w, b, ea, ef)
    return la, lf, va, vf


@jax.jit
def kernel(query_features, W_attn, b_attn, W_ffn, b_ffn,
           E_attn_vis, E_attn_text, E_attn_general,
           E_ffn_vis, E_ffn_text, E_ffn_general):
    w = jnp.concatenate([W_attn, W_ffn], axis=0)
    b = jnp.concatenate([b_attn, b_ffn])[None, :]
    ea = jnp.concatenate([E_attn_vis, E_attn_text, E_attn_general], axis=0)
    ef = jnp.concatenate([E_ffn_vis, E_ffn_text, E_ffn_general], axis=0)

    devs = jax.devices()
    ndev = 1
    for n in (2, 4, 8):
        if len(devs) >= n and (B // n) % BLK == 0:
            ndev = n
    mesh = Mesh(np.array(devs[:ndev]), ("d",))
    rep = P(None, None)
    f = jax.shard_map(
        _run_shard, mesh=mesh,
        in_specs=(P("d", None), rep, rep, rep, rep),
        out_specs=(P("d", None), P("d", None), P("d", None), P("d", None)),
        check_vma=False,
    )
    la, lf, va, vf = f(query_features, w, b, ea, ef)
    return (va, vf, la, lf)
